# Initial kernel scaffold; baseline (speedup 1.0000x reference)
import jax
import jax.numpy as jnp
from jax.experimental import pallas as pl


def kernel(x, W):
    raise NotImplementedError("write your pallas kernel here")


# speedup vs baseline: 1.8977x; 1.8977x over previous
import jax
import jax.numpy as jnp
from jax import lax
from jax.experimental import pallas as pl
from jax.experimental.pallas import tpu as pltpu

_HIDDEN = 2048
_E = 64
_T = 8192
_BT = 1024
_NBLK = _T // _BT
_TOL = 1e-4
_EPS = 1e-8


def _router_kernel(x_ref, wt_ref, scores_ref, idx_ref, logits_ref):
    i = pl.program_id(0)
    logits_ref[pl.ds(i * _BT, _BT), :] = jnp.dot(
        x_ref[...], wt_ref[...], preferred_element_type=jnp.float32
    )

    @pl.when(i == _NBLK - 1)
    def _finish():
        logits = logits_ref[...]
        cost = jnp.exp(logits)

        def cond(state):
            return state[2] > _TOL

        def body(state):
            d1c, _, _ = state
            t0 = jnp.sum(d1c * cost, axis=1, keepdims=True)
            d0n = (1.0 / _T) * (1.0 / (t0 + _EPS))
            s1 = jnp.sum(d0n * cost, axis=0, keepdims=True)
            d1n = (1.0 / _E) * (1.0 / (s1 + _EPS))
            err = jnp.mean(jnp.abs(d1c - d1n))
            return (d1n, d0n, err)

        init = (
            jnp.ones((1, _E), dtype=jnp.float32),
            jnp.zeros((_T, 1), dtype=jnp.float32),
            jnp.float32(1e9),
        )
        d1f, d0f, _ = lax.while_loop(cond, body, init)

        norm = (d1f * cost) * d0f
        mx = jnp.max(norm, axis=1, keepdims=True)
        iota = lax.broadcasted_iota(jnp.int32, (_T, _E), 1)
        idx = jnp.min(jnp.where(norm == mx, iota, _E), axis=1, keepdims=True)
        act = jax.nn.sigmoid(logits)
        scores_ref[...] = jnp.sum(
            jnp.where(iota == idx, act, 0.0), axis=1, keepdims=True
        )
        idx_ref[...] = idx


def kernel(x, W):
    x2 = x.reshape(-1, x.shape[-1])
    wt = W.T
    scores, idx = pl.pallas_call(
        _router_kernel,
        grid=(_NBLK,),
        in_specs=[
            pl.BlockSpec((_BT, _HIDDEN), lambda i: (i, 0)),
            pl.BlockSpec((_HIDDEN, _E), lambda i: (0, 0)),
        ],
        out_specs=[
            pl.BlockSpec((_T, 1), lambda i: (0, 0)),
            pl.BlockSpec((_T, 1), lambda i: (0, 0)),
        ],
        out_shape=[
            jax.ShapeDtypeStruct((_T, 1), jnp.float32),
            jax.ShapeDtypeStruct((_T, 1), jnp.int32),
        ],
        scratch_shapes=[pltpu.VMEM((_T, _E), jnp.float32)],
        compiler_params=pltpu.CompilerParams(
            dimension_semantics=("arbitrary",),
        ),
    )(x2, wt)
    return (scores, idx)
